# Initial kernel scaffold; baseline (speedup 1.0000x reference)
import jax
import jax.numpy as jnp
from jax.experimental import pallas as pl


def kernel(x, emb_weight):
    raise NotImplementedError("write your pallas kernel here")


# speedup vs baseline: 1.6992x; 1.6992x over previous
import jax
import jax.numpy as jnp
from jax.experimental import pallas as pl

T_BLK = 512


def _add_kernel(x_ref, emb_ref, out_ref):
    out_ref[...] = x_ref[...] + emb_ref[...]


def kernel(x, emb_weight):
    B, T, D = x.shape
    grid = (T // T_BLK, B)
    return pl.pallas_call(
        _add_kernel,
        grid=grid,
        in_specs=[
            pl.BlockSpec((1, T_BLK, D), lambda t, b: (b, t, 0)),
            pl.BlockSpec((T_BLK, D), lambda t, b: (t, 0)),
        ],
        out_specs=pl.BlockSpec((1, T_BLK, D), lambda t, b: (b, t, 0)),
        out_shape=jax.ShapeDtypeStruct((B, T, D), x.dtype),
    )(x, emb_weight[:T])
